# baseline (speedup 1.0000x reference)
import jax
import jax.numpy as jnp
from jax.experimental import pallas as pl
from jax.experimental.pallas import tpu as pltpu

_NBUF = 4
_BM = 200


def _gcn_kernel(adj_hbm, x_ref, w_ref, b_ref, out_ref, abuf, s_ref, sems):
    i = pl.program_id(0)
    nblk = pl.num_programs(0)

    @pl.when(i == 0)
    def _():
        for j in range(_NBUF - 1):
            pltpu.make_async_copy(
                adj_hbm.at[pl.ds(j * _BM, _BM), :], abuf.at[j], sems.at[j]
            ).start()
        s_ref[...] = jnp.dot(x_ref[...], w_ref[...],
                             preferred_element_type=jnp.float32)

    nxt = i + _NBUF - 1

    @pl.when(nxt < nblk)
    def _():
        slot = jax.lax.rem(nxt, _NBUF)
        pltpu.make_async_copy(
            adj_hbm.at[pl.ds(nxt * _BM, _BM), :], abuf.at[slot], sems.at[slot]
        ).start()

    slot = jax.lax.rem(i, _NBUF)
    pltpu.make_async_copy(
        adj_hbm.at[pl.ds(i * _BM, _BM), :], abuf.at[slot], sems.at[slot]
    ).wait()
    acc = jnp.dot(abuf[slot], s_ref[...], preferred_element_type=jnp.float32)
    out_ref[...] = jnp.maximum(acc + b_ref[...], 0.0)


def kernel(x, adj, W, b):
    n, d_in = x.shape
    d_out = W.shape[1]
    b2 = b.reshape(1, d_out)
    out = pl.pallas_call(
        _gcn_kernel,
        grid=(n // _BM,),
        in_specs=[
            pl.BlockSpec(memory_space=pltpu.MemorySpace.HBM),
            pl.BlockSpec((n, d_in), lambda i: (0, 0)),
            pl.BlockSpec((d_in, d_out), lambda i: (0, 0)),
            pl.BlockSpec((1, d_out), lambda i: (0, 0)),
        ],
        out_specs=pl.BlockSpec((_BM, d_out), lambda i: (i, 0)),
        out_shape=jax.ShapeDtypeStruct((n, d_out), jnp.float32),
        scratch_shapes=[
            pltpu.VMEM((_NBUF, _BM, n), jnp.float32),
            pltpu.VMEM((n, d_out), jnp.float32),
            pltpu.SemaphoreType.DMA((_NBUF,)),
        ],
    )(adj, x, W, b2)
    return out


# speedup vs baseline: 1.0119x; 1.0119x over previous
import jax
import jax.numpy as jnp
from jax.experimental import pallas as pl


def _gcn_kernel(adj_ref, x_ref, w_ref, b_ref, out_ref):
    t = jnp.dot(adj_ref[...], x_ref[...], preferred_element_type=jnp.float32)
    acc = jnp.dot(t, w_ref[...], preferred_element_type=jnp.float32)
    out_ref[...] = jnp.maximum(acc + b_ref[...], 0.0)


def kernel(x, adj, W, b):
    n, d_in = x.shape
    d_out = W.shape[1]
    bm = 400
    b2 = b.reshape(1, d_out)
    out = pl.pallas_call(
        _gcn_kernel,
        grid=(n // bm,),
        in_specs=[
            pl.BlockSpec((bm, n), lambda i: (i, 0)),
            pl.BlockSpec((n, d_in), lambda i: (0, 0)),
            pl.BlockSpec((d_in, d_out), lambda i: (0, 0)),
            pl.BlockSpec((1, d_out), lambda i: (0, 0)),
        ],
        out_specs=pl.BlockSpec((bm, d_out), lambda i: (i, 0)),
        out_shape=jax.ShapeDtypeStruct((n, d_out), jnp.float32),
    )(adj, x, W, b2)
    return out
